# Initial kernel scaffold; baseline (speedup 1.0000x reference)
import jax
import jax.numpy as jnp
from jax.experimental import pallas as pl


def kernel(x, edge_index, batch, W1, b1, W2, b2, Wfc, bfc):
    raise NotImplementedError("write your pallas kernel here")


# speedup vs baseline: 49.1427x; 49.1427x over previous
import jax
import jax.numpy as jnp
from jax import lax
from jax.experimental import pallas as pl
from jax.experimental.pallas import tpu as pltpu
from jax.experimental.pallas import tpu_sc as plsc

N = 10000
E = 320000
D = 128
H = 64
C = 10
B = 128

NH = N // 2
NC = 2
NS = 16
NW = NC * NS
K = 128
NCHUNK = E // K
BUFC = 88
BUFD = 168
NP = 10240
RPT = NP // NS
DEGW = 16
PB = 160
NBUF = 8

BLK = 1000
GRID = NH // BLK


def _chunk_range(idx, nparts, buf):
  s = (NCHUNK * idx) // nparts
  e = (NCHUNK * (idx + 1)) // nparts
  base = jnp.minimum(s - lax.rem(s, 8), NCHUNK - buf)
  return s - base, e - s, base


def _deg_body(pdst2, ones_hbm, zeros_hbm, deg_out, didx, ones_v, dsem, degl,
              dbuf, dacc):
  cid = lax.axis_index("c")
  sid = lax.axis_index("s")
  w = sid * NC + cid
  pltpu.sync_copy(zeros_hbm, dacc.at[pl.ds(sid * RPT, RPT)])
  pltpu.sync_copy(ones_hbm, ones_v)
  off, cnt, base = _chunk_range(sid, NS, BUFD)
  pltpu.sync_copy(pdst2.at[pl.ds(base, BUFD)], didx)
  plsc.subcore_barrier()

  G = 20

  def group(g, carry):
    def fire(j, c):
      @pl.when(g * G + j < cnt)
      def _():
        pltpu.async_copy(ones_v, dacc.at[didx.at[off + g * G + j]], dsem,
                         add=True)
      return c

    lax.fori_loop(0, G, fire, 0)

    def drain(j, c):
      @pl.when(g * G + j < cnt)
      def _():
        pltpu.make_async_copy(ones_v, dacc.at[didx.at[off + g * G + j]],
                              dsem).wait()
      return c

    lax.fori_loop(0, G, drain, 0)
    return carry

  lax.fori_loop(0, 8, group, 0)
  plsc.subcore_barrier()

  pstart = jnp.minimum(PB * w, NH - PB)
  pltpu.sync_copy(dacc.at[pl.ds(2 * pstart, 2 * PB)], degl)

  def finrow(i, carry):
    ylo = degl[2 * i, :]
    yhi = degl[2 * i + 1, :]
    for k2 in range(4):
      dbuf[i, pl.ds(16 * k2, 16)] = ylo
      dbuf[i, pl.ds(64 + 16 * k2, 16)] = yhi
    return carry

  lax.fori_loop(0, PB, finrow, 0)
  pltpu.sync_copy(dbuf, deg_out.at[pl.ds(pstart, PB)])


def _spmv_body(hs, psrc2, pdst2, zeros_hbm, out, sidx, didx, rows, gsem, ssem,
               acc):
  cid = lax.axis_index("c")
  sid = lax.axis_index("s")
  w = sid * NC + cid
  pltpu.sync_copy(zeros_hbm, acc.at[pl.ds(sid * RPT, RPT)])
  off, cnt, base = _chunk_range(w, NW, BUFC)
  pltpu.sync_copy(psrc2.at[pl.ds(base, BUFC)], sidx)
  pltpu.sync_copy(pdst2.at[pl.ds(base, BUFC)], didx)
  plsc.subcore_barrier()

  for b in range(NBUF):
    @pl.when(b < cnt)
    def _(b=b):
      pltpu.async_copy(hs.at[sidx.at[off + b]], rows.at[b], gsem.at[b])

  def body(j, carry):
    b = lax.rem(j, NBUF)
    pltpu.make_async_copy(hs.at[sidx.at[off + j]], rows.at[b],
                          gsem.at[b]).wait()
    pltpu.async_copy(rows.at[b], acc.at[didx.at[off + j]], ssem.at[b],
                     add=True)

    @pl.when((j >= 1) & (j - 1 + NBUF < cnt))
    def _():
      bp = lax.rem(j - 1, NBUF)
      pltpu.make_async_copy(rows.at[bp], acc.at[didx.at[off + j - 1]],
                            ssem.at[bp]).wait()
      pltpu.async_copy(hs.at[sidx.at[off + j - 1 + NBUF]], rows.at[bp],
                       gsem.at[bp])
    return carry

  lax.fori_loop(0, cnt, body, 0)
  for b in range(NBUF):
    @pl.when(b < cnt)
    def _(b=b):
      pltpu.make_async_copy(rows.at[b], acc.at[didx.at[off]], ssem.at[b]).wait()
  plsc.subcore_barrier()
  pltpu.sync_copy(acc.at[pl.ds(sid * RPT, RPT)],
                  out.at[cid, pl.ds(sid * RPT, RPT)])


_deg_call = pl.kernel(
    _deg_body,
    out_type=jax.ShapeDtypeStruct((NH, D), jnp.float32),
    mesh=plsc.VectorSubcoreMesh(core_axis_name="c", subcore_axis_name="s"),
    scratch_types=[
        pltpu.VMEM((BUFD, K), jnp.int32),
        pltpu.VMEM((K, DEGW), jnp.float32),
        pltpu.SemaphoreType.DMA,
        pltpu.VMEM((2 * PB, DEGW), jnp.float32),
        pltpu.VMEM((PB, D), jnp.float32),
        pltpu.VMEM_SHARED((NP, DEGW), jnp.float32),
    ],
    compiler_params=pltpu.CompilerParams(use_tc_tiling_on_sc=False),
)

_spmv_call = pl.kernel(
    _spmv_body,
    out_type=jax.ShapeDtypeStruct((NC, NP, H), jnp.float32),
    mesh=plsc.VectorSubcoreMesh(core_axis_name="c", subcore_axis_name="s"),
    scratch_types=[
        pltpu.VMEM((BUFC, K), jnp.int32),
        pltpu.VMEM((BUFC, K), jnp.int32),
        pltpu.VMEM((NBUF, K, H), jnp.float32),
        pltpu.SemaphoreType.DMA((NBUF,)),
        pltpu.SemaphoreType.DMA((NBUF,)),
        pltpu.VMEM_SHARED((NP, H), jnp.float32),
    ],
    compiler_params=pltpu.CompilerParams(use_tc_tiling_on_sc=False),
)

_DOT = dict(preferred_element_type=jnp.float32, precision=lax.Precision.DEFAULT)


def _mm1_body(xlo, xhi, degp, w1b, hs_out, dinv_out):
  dinv = lax.rsqrt(degp[...] + 1.0)
  xp = jnp.concatenate([xlo[...], xhi[...]], axis=1)
  hs_out[...] = jnp.dot(xp, w1b[...], **_DOT) * dinv
  dinv_out[...] = dinv


def _mm2_body(part, hs, dinvp, w2b, b1w, hs2_out):
  p = part[...]
  dv = dinvp[...]
  z = (p[0] + p[1] + hs[...]) * dv + b1w[...]
  h1 = jnp.maximum(z, 0.0)
  hs2_out[...] = jnp.dot(h1, w2b[...], **_DOT) * dv


def _pool_body(part, hs2, dinvp, b2w, batch4, wfc, bfc, out, accs, accc):
  i = pl.program_id(0)

  @pl.when(i == 0)
  def _():
    accs[...] = jnp.zeros((B, H), jnp.float32)
    accc[...] = jnp.zeros((B, 8), jnp.float32)

  p = part[...]
  h2 = jnp.maximum((p[0] + p[1] + hs2[...]) * dinvp[...] + b2w[...], 0.0)
  cdims = (((0,), (0,)), ((), ()))
  ones8 = jnp.ones((BLK, 8), jnp.float32)
  iota = lax.broadcasted_iota(jnp.int32, (BLK, B), 1)
  for half in range(2):
    bt = batch4[half, 0, :, :]
    oneh = (bt == iota).astype(jnp.float32)
    h2h = h2[:, half * H:(half + 1) * H]
    accs[...] += lax.dot_general(oneh, h2h, cdims, **_DOT)
    accc[...] += lax.dot_general(oneh, ones8, cdims, **_DOT)

  @pl.when(i == pl.num_programs(0) - 1)
  def _():
    cnt = jnp.maximum(accc[:, 0:1], 1.0)
    pooled = accs[...] / cnt
    logits = jnp.dot(pooled, wfc[...], **_DOT) + bfc[...]
    m = jnp.max(logits, axis=1, keepdims=True)
    e = jnp.exp(logits - m)
    out[...] = logits - m - jnp.log(jnp.sum(e, axis=1, keepdims=True))


_mm1_call = pl.pallas_call(
    _mm1_body,
    grid=(GRID,),
    in_specs=[
        pl.BlockSpec((BLK, D), lambda i: (i, 0)),
        pl.BlockSpec((BLK, D), lambda i: (i + GRID, 0)),
        pl.BlockSpec((BLK, D), lambda i: (i, 0)),
        pl.BlockSpec((2 * D, D), lambda i: (0, 0)),
    ],
    out_specs=[
        pl.BlockSpec((BLK, D), lambda i: (i, 0)),
        pl.BlockSpec((BLK, D), lambda i: (i, 0)),
    ],
    out_shape=[
        jax.ShapeDtypeStruct((NH, D), jnp.float32),
        jax.ShapeDtypeStruct((NH, D), jnp.float32),
    ],
)

_mm2_call = pl.pallas_call(
    _mm2_body,
    grid=(GRID,),
    in_specs=[
        pl.BlockSpec((NC, BLK, D), lambda i: (0, i, 0)),
        pl.BlockSpec((BLK, D), lambda i: (i, 0)),
        pl.BlockSpec((BLK, D), lambda i: (i, 0)),
        pl.BlockSpec((D, D), lambda i: (0, 0)),
        pl.BlockSpec((1, D), lambda i: (0, 0)),
    ],
    out_specs=pl.BlockSpec((BLK, D), lambda i: (i, 0)),
    out_shape=jax.ShapeDtypeStruct((NH, D), jnp.float32),
)

_pool_call = pl.pallas_call(
    _pool_body,
    grid=(GRID,),
    in_specs=[
        pl.BlockSpec((NC, BLK, D), lambda i: (0, i, 0)),
        pl.BlockSpec((BLK, D), lambda i: (i, 0)),
        pl.BlockSpec((BLK, D), lambda i: (i, 0)),
        pl.BlockSpec((1, D), lambda i: (0, 0)),
        pl.BlockSpec((2, 1, BLK, 1), lambda i: (0, i, 0, 0)),
        pl.BlockSpec((H, C), lambda i: (0, 0)),
        pl.BlockSpec((1, C), lambda i: (0, 0)),
    ],
    out_specs=pl.BlockSpec((B, C), lambda i: (0, 0)),
    out_shape=jax.ShapeDtypeStruct((B, C), jnp.float32),
    scratch_shapes=[
        pltpu.VMEM((B, H), jnp.float32),
        pltpu.VMEM((B, 8), jnp.float32),
    ],
)


def _blockdiag2(w):
  k, m = w.shape
  z = jnp.zeros((k, m), w.dtype)
  return jnp.concatenate(
      [jnp.concatenate([w, z], axis=1), jnp.concatenate([z, w], axis=1)],
      axis=0)


def kernel(x, edge_index, batch, W1, b1, W2, b2, Wfc, bfc):
  ei = edge_index.astype(jnp.int32)
  perm = lambda v: jnp.where(v < NH, v * 2, v * 2 - (N - 1))
  psrc2 = perm(ei[0]).reshape(NCHUNK, K)
  pdst2 = perm(ei[1]).reshape(NCHUNK, K)
  ones_hbm = jnp.ones((K, DEGW), jnp.float32)
  zeros16 = jnp.zeros((RPT, DEGW), jnp.float32)
  zerosH = jnp.zeros((RPT, H), jnp.float32)
  batch4 = batch.astype(jnp.int32).reshape(2, GRID, BLK, 1)
  w1b = _blockdiag2(W1)
  w2b = _blockdiag2(W2)
  b1w = jnp.concatenate([b1, b1]).reshape(1, D)
  b2w = jnp.concatenate([b2, b2]).reshape(1, D)

  degp = _deg_call(pdst2, ones_hbm, zeros16)
  hs1, dinvp = _mm1_call(x, x, degp, w1b)
  part1 = _spmv_call(hs1.reshape(N, H), psrc2, pdst2, zerosH)
  hs2 = _mm2_call(part1.reshape(NC, NP // 2, D), hs1, dinvp, w2b, b1w)
  part2 = _spmv_call(hs2.reshape(N, H), psrc2, pdst2, zerosH)
  ls = _pool_call(part2.reshape(NC, NP // 2, D), hs2, dinvp, b2w, batch4, Wfc,
                  bfc.reshape(1, C))
  return (ls, jnp.array(1))


# speedup vs baseline: 54.2141x; 1.1032x over previous
import jax
import jax.numpy as jnp
from jax import lax
from jax.experimental import pallas as pl
from jax.experimental.pallas import tpu as pltpu
from jax.experimental.pallas import tpu_sc as plsc

N = 10000
E = 320000
D = 128
H = 64
C = 10
B = 128

NH = N // 2
NC = 2
NS = 16
NW = NC * NS
K = 128
NCHUNK = E // K
BUFC = 88
BUFD = 168
NP = 10240
RPT = NP // NS
DEGW = 16
PB = 160
NBUF = 8

BLK = 1000
GRID = NH // BLK


def _chunk_range(idx, nparts, buf):
  s = (NCHUNK * idx) // nparts
  e = (NCHUNK * (idx + 1)) // nparts
  base = jnp.minimum(s - lax.rem(s, 8), NCHUNK - buf)
  return s - base, e - s, base


def _deg_body(pdst2, ones_hbm, zeros_hbm, deg_out, didx, ones_v, dsem, degl,
              dbuf, dacc):
  cid = lax.axis_index("c")
  sid = lax.axis_index("s")
  w = sid * NC + cid
  pltpu.sync_copy(zeros_hbm, dacc.at[pl.ds(sid * RPT, RPT)])
  pltpu.sync_copy(ones_hbm, ones_v)
  off, cnt, base = _chunk_range(sid, NS, BUFD)
  pltpu.sync_copy(pdst2.at[pl.ds(base, BUFD)], didx)
  plsc.subcore_barrier()

  G = 20

  def group(g, carry):
    def fire(j, c):
      @pl.when(g * G + j < cnt)
      def _():
        pltpu.async_copy(ones_v, dacc.at[didx.at[off + g * G + j]], dsem,
                         add=True)
      return c

    lax.fori_loop(0, G, fire, 0)

    def drain(j, c):
      @pl.when(g * G + j < cnt)
      def _():
        pltpu.make_async_copy(ones_v, dacc.at[didx.at[off + g * G + j]],
                              dsem).wait()
      return c

    lax.fori_loop(0, G, drain, 0)
    return carry

  lax.fori_loop(0, 8, group, 0)
  plsc.subcore_barrier()

  pstart = jnp.minimum(PB * w, NH - PB)
  pltpu.sync_copy(dacc.at[pl.ds(2 * pstart, 2 * PB)], degl)

  def finrow(i, carry):
    ylo = degl[2 * i, :]
    yhi = degl[2 * i + 1, :]
    for k2 in range(4):
      dbuf[i, pl.ds(16 * k2, 16)] = ylo
      dbuf[i, pl.ds(64 + 16 * k2, 16)] = yhi
    return carry

  lax.fori_loop(0, PB, finrow, 0)
  pltpu.sync_copy(dbuf, deg_out.at[pl.ds(pstart, PB)])


def _spmv_body(hs, psrc2, pdst2, zeros_hbm, out, sidx, didx, rows, gsem, ssem,
               acc):
  cid = lax.axis_index("c")
  sid = lax.axis_index("s")
  w = sid * NC + cid
  pltpu.sync_copy(zeros_hbm, acc.at[pl.ds(sid * RPT, RPT)])
  off, cnt, base = _chunk_range(w, NW, BUFC)
  pltpu.sync_copy(psrc2.at[pl.ds(base, BUFC)], sidx)
  pltpu.sync_copy(pdst2.at[pl.ds(base, BUFC)], didx)
  plsc.subcore_barrier()

  for b in range(NBUF):
    @pl.when(b < cnt)
    def _(b=b):
      pltpu.async_copy(hs.at[sidx.at[off + b]], rows.at[b], gsem.at[b])

  def body(j, carry):
    b = lax.rem(j, NBUF)
    pltpu.make_async_copy(hs.at[sidx.at[off + j]], rows.at[b],
                          gsem.at[b]).wait()
    pltpu.async_copy(rows.at[b], acc.at[didx.at[off + j]], ssem.at[b],
                     add=True)

    @pl.when((j >= 1) & (j - 1 + NBUF < cnt))
    def _():
      bp = lax.rem(j - 1, NBUF)
      pltpu.make_async_copy(rows.at[bp], acc.at[didx.at[off + j - 1]],
                            ssem.at[bp]).wait()
      pltpu.async_copy(hs.at[sidx.at[off + j - 1 + NBUF]], rows.at[bp],
                       gsem.at[bp])
    return carry

  lax.fori_loop(0, cnt, body, 0)
  for b in range(NBUF):
    @pl.when(b < cnt)
    def _(b=b):
      pltpu.make_async_copy(rows.at[b], acc.at[didx.at[off]], ssem.at[b]).wait()
  plsc.subcore_barrier()
  pltpu.sync_copy(acc.at[pl.ds(sid * RPT, RPT)],
                  out.at[cid, pl.ds(sid * RPT, RPT)])


_deg_call = pl.kernel(
    _deg_body,
    out_type=jax.ShapeDtypeStruct((NH, D), jnp.float32),
    mesh=plsc.VectorSubcoreMesh(core_axis_name="c", subcore_axis_name="s"),
    scratch_types=[
        pltpu.VMEM((BUFD, K), jnp.int32),
        pltpu.VMEM((K, DEGW), jnp.float32),
        pltpu.SemaphoreType.DMA,
        pltpu.VMEM((2 * PB, DEGW), jnp.float32),
        pltpu.VMEM((PB, D), jnp.float32),
        pltpu.VMEM_SHARED((NP, DEGW), jnp.float32),
    ],
    compiler_params=pltpu.CompilerParams(use_tc_tiling_on_sc=False),
)

_spmv_call = pl.kernel(
    _spmv_body,
    out_type=jax.ShapeDtypeStruct((NC, NP, H), jnp.float32),
    mesh=plsc.VectorSubcoreMesh(core_axis_name="c", subcore_axis_name="s"),
    scratch_types=[
        pltpu.VMEM((BUFC, K), jnp.int32),
        pltpu.VMEM((BUFC, K), jnp.int32),
        pltpu.VMEM((NBUF, K, H), jnp.float32),
        pltpu.SemaphoreType.DMA((NBUF,)),
        pltpu.SemaphoreType.DMA((NBUF,)),
        pltpu.VMEM_SHARED((NP, H), jnp.float32),
    ],
    compiler_params=pltpu.CompilerParams(use_tc_tiling_on_sc=False),
)

_DOT = dict(preferred_element_type=jnp.float32, precision=lax.Precision.DEFAULT)


def _eprep_body(ei, psrc_out, pdst_out):
  v = ei[...]
  p = v + v - jnp.where(v < NH, 0, N - 1)
  psrc_out[...] = p[0]
  pdst_out[...] = p[1]


_eprep_call = pl.pallas_call(
    _eprep_body,
    grid=(1,),
    in_specs=[pl.BlockSpec((2, NCHUNK, K), lambda i: (0, 0, 0))],
    out_specs=[
        pl.BlockSpec((NCHUNK, K), lambda i: (0, 0)),
        pl.BlockSpec((NCHUNK, K), lambda i: (0, 0)),
    ],
    out_shape=[
        jax.ShapeDtypeStruct((NCHUNK, K), jnp.int32),
        jax.ShapeDtypeStruct((NCHUNK, K), jnp.int32),
    ],
)


def _mm1_body(xlo, xhi, degp, w1b, hs_out, dinv_out):
  dinv = lax.rsqrt(degp[...] + 1.0)
  xp = jnp.concatenate([xlo[...], xhi[...]], axis=1)
  hs_out[...] = jnp.dot(xp, w1b[...], **_DOT) * dinv
  dinv_out[...] = dinv


def _mm2_body(part, hs, dinvp, w2b, b1w, hs2_out):
  p = part[...]
  dv = dinvp[...]
  z = (p[0] + p[1] + hs[...]) * dv + b1w[...]
  h1 = jnp.maximum(z, 0.0)
  hs2_out[...] = jnp.dot(h1, w2b[...], **_DOT) * dv


def _pool_body(part, hs2, dinvp, b2w, batch4, wfc, bfc, out, accs, accc):
  i = pl.program_id(0)

  @pl.when(i == 0)
  def _():
    accs[...] = jnp.zeros((B, H), jnp.float32)
    accc[...] = jnp.zeros((B, 8), jnp.float32)

  p = part[...]
  h2 = jnp.maximum((p[0] + p[1] + hs2[...]) * dinvp[...] + b2w[...], 0.0)
  cdims = (((0,), (0,)), ((), ()))
  ones8 = jnp.ones((BLK, 8), jnp.float32)
  iota = lax.broadcasted_iota(jnp.int32, (BLK, B), 1)
  for half in range(2):
    bt = batch4[half, 0, :, :]
    oneh = (bt == iota).astype(jnp.float32)
    h2h = h2[:, half * H:(half + 1) * H]
    accs[...] += lax.dot_general(oneh, h2h, cdims, **_DOT)
    accc[...] += lax.dot_general(oneh, ones8, cdims, **_DOT)

  @pl.when(i == pl.num_programs(0) - 1)
  def _():
    cnt = jnp.maximum(accc[:, 0:1], 1.0)
    pooled = accs[...] / cnt
    logits = jnp.dot(pooled, wfc[...], **_DOT) + bfc[...]
    m = jnp.max(logits, axis=1, keepdims=True)
    e = jnp.exp(logits - m)
    out[...] = logits - m - jnp.log(jnp.sum(e, axis=1, keepdims=True))


_mm1_call = pl.pallas_call(
    _mm1_body,
    grid=(GRID,),
    in_specs=[
        pl.BlockSpec((BLK, D), lambda i: (i, 0)),
        pl.BlockSpec((BLK, D), lambda i: (i + GRID, 0)),
        pl.BlockSpec((BLK, D), lambda i: (i, 0)),
        pl.BlockSpec((2 * D, D), lambda i: (0, 0)),
    ],
    out_specs=[
        pl.BlockSpec((BLK, D), lambda i: (i, 0)),
        pl.BlockSpec((BLK, D), lambda i: (i, 0)),
    ],
    out_shape=[
        jax.ShapeDtypeStruct((NH, D), jnp.float32),
        jax.ShapeDtypeStruct((NH, D), jnp.float32),
    ],
)

_mm2_call = pl.pallas_call(
    _mm2_body,
    grid=(GRID,),
    in_specs=[
        pl.BlockSpec((NC, BLK, D), lambda i: (0, i, 0)),
        pl.BlockSpec((BLK, D), lambda i: (i, 0)),
        pl.BlockSpec((BLK, D), lambda i: (i, 0)),
        pl.BlockSpec((D, D), lambda i: (0, 0)),
        pl.BlockSpec((1, D), lambda i: (0, 0)),
    ],
    out_specs=pl.BlockSpec((BLK, D), lambda i: (i, 0)),
    out_shape=jax.ShapeDtypeStruct((NH, D), jnp.float32),
)

_pool_call = pl.pallas_call(
    _pool_body,
    grid=(GRID,),
    in_specs=[
        pl.BlockSpec((NC, BLK, D), lambda i: (0, i, 0)),
        pl.BlockSpec((BLK, D), lambda i: (i, 0)),
        pl.BlockSpec((BLK, D), lambda i: (i, 0)),
        pl.BlockSpec((1, D), lambda i: (0, 0)),
        pl.BlockSpec((2, 1, BLK, 1), lambda i: (0, i, 0, 0)),
        pl.BlockSpec((H, C), lambda i: (0, 0)),
        pl.BlockSpec((1, C), lambda i: (0, 0)),
    ],
    out_specs=pl.BlockSpec((B, C), lambda i: (0, 0)),
    out_shape=jax.ShapeDtypeStruct((B, C), jnp.float32),
    scratch_shapes=[
        pltpu.VMEM((B, H), jnp.float32),
        pltpu.VMEM((B, 8), jnp.float32),
    ],
)


def _blockdiag2(w):
  k, m = w.shape
  z = jnp.zeros((k, m), w.dtype)
  return jnp.concatenate(
      [jnp.concatenate([w, z], axis=1), jnp.concatenate([z, w], axis=1)],
      axis=0)


def kernel(x, edge_index, batch, W1, b1, W2, b2, Wfc, bfc):
  ei3 = edge_index.astype(jnp.int32).reshape(2, NCHUNK, K)
  psrc2, pdst2 = _eprep_call(ei3)
  ones_hbm = jnp.ones((K, DEGW), jnp.float32)
  zeros16 = jnp.zeros((RPT, DEGW), jnp.float32)
  zerosH = jnp.zeros((RPT, H), jnp.float32)
  batch4 = batch.astype(jnp.int32).reshape(2, GRID, BLK, 1)
  w1b = _blockdiag2(W1)
  w2b = _blockdiag2(W2)
  b1w = jnp.concatenate([b1, b1]).reshape(1, D)
  b2w = jnp.concatenate([b2, b2]).reshape(1, D)

  degp = _deg_call(pdst2, ones_hbm, zeros16)
  hs1, dinvp = _mm1_call(x, x, degp, w1b)
  part1 = _spmv_call(hs1.reshape(N, H), psrc2, pdst2, zerosH)
  hs2 = _mm2_call(part1.reshape(NC, NP // 2, D), hs1, dinvp, w2b, b1w)
  part2 = _spmv_call(hs2.reshape(N, H), psrc2, pdst2, zerosH)
  ls = _pool_call(part2.reshape(NC, NP // 2, D), hs2, dinvp, b2w, batch4, Wfc,
                  bfc.reshape(1, C))
  return (ls, jnp.array(1))
